# baseline (speedup 1.0000x reference)
import functools

import jax
import jax.numpy as jnp
import numpy as np
from jax import lax
from jax.experimental import pallas as pl
from jax.experimental.pallas import tpu as pltpu
from jax.experimental.pallas import tpu_sc as plsc

_B = 16
_N = 4096
_D = 64
_C = 1024
_K = 32
_R2 = 0.2 * 0.2
_IN = 3 + _D
_M = _B * _C * _K

_TM = 8192
_GM = _M // _TM
_TC = _TM // _K
_EPS = 1e-5

_NW = 32
_ROWS_W = _M // _NW
_GCHUNK = 512

_INTERPRET = False


def _fps_body(ct_ref, far0_ref, out_ref):
    ct = ct_ref[...]
    lin = jax.lax.broadcasted_iota(jnp.int32, (_B, _N), 1)
    big = jnp.int32(2 ** 30)

    offs = jax.lax.broadcasted_iota(jnp.int32, (1, _B), 1) * _N

    def body(i, carry):
        dist, far = carry
        out_ref[pl.ds(i, 1), :] = far.reshape(1, _B) + offs
        onehot = (lin == far).astype(jnp.float32)
        cd = [jnp.sum(onehot * ct[:, d, :], axis=1, keepdims=True)[:, None, :]
              for d in range(3)]
        dd = ct - jnp.concatenate(cd, axis=1)
        d2 = jnp.sum(dd * dd, axis=1)
        dist = jnp.minimum(dist, d2)
        m = jnp.max(dist, axis=1, keepdims=True)
        far = jnp.min(jnp.where(dist == m, lin, big), axis=1, keepdims=True)
        return dist, far

    jax.lax.fori_loop(
        0, _C, body,
        (jnp.full((_B, _N), jnp.float32(jnp.inf)), far0_ref[...]))


def _fps_idx(coords):
    farthest0 = jax.random.randint(jax.random.key(42), (_B,), 0, _N, dtype=jnp.int32)
    out = pl.pallas_call(
        _fps_body,
        grid=(1,),
        in_specs=[pl.BlockSpec((_B, 3, _N), lambda i: (0, 0, 0)),
                  pl.BlockSpec((_B, 1), lambda i: (0, 0))],
        out_specs=pl.BlockSpec((_C, _B), lambda i: (0, 0)),
        out_shape=jax.ShapeDtypeStruct((_C, _B), jnp.int32),
        interpret=_INTERPRET,
    )(jnp.transpose(coords, (0, 2, 1)), farthest0[:, None])
    return out.T.reshape(_B * _C)


_TCB = 256


def _topk_body(cent_ref, ct_ref, out_ref):
    ct = ct_ref[0]
    cent = cent_ref[...]
    x2 = jnp.sum(ct * ct, axis=0, keepdims=True)
    c2 = jnp.sum(cent * cent, axis=1, keepdims=True)
    dots = jnp.dot(cent, ct, preferred_element_type=jnp.float32)
    d2 = jnp.maximum(c2 + x2 - 2.0 * dots, 0.0)
    vals0 = jnp.where(d2 <= _R2, d2, jnp.float32(1e30))
    lin = jax.lax.broadcasted_iota(jnp.int32, (_TCB, _N), 1)
    kl = jax.lax.broadcasted_iota(jnp.int32, (_TCB, _K), 1)
    big = jnp.int32(2 ** 30)

    def body(k, carry):
        vals, acc, prev = carry
        vals = jnp.where(lin == prev, jnp.float32(1e31), vals)
        m = jnp.min(vals, axis=1, keepdims=True)
        idx = jnp.min(jnp.where(vals == m, lin, big), axis=1, keepdims=True)
        acc = jnp.where(kl == k, idx, acc)
        return vals, acc, idx

    _, acc, _ = jax.lax.fori_loop(
        0, _K, body,
        (vals0, jnp.zeros((_TCB, _K), jnp.int32),
         jnp.full((_TCB, 1), jnp.int32(-1))))
    out_ref[...] = acc + pl.program_id(0) * _N


def _ball_topk(cent_flat, coords_t):
    jc = _C // _TCB
    return pl.pallas_call(
        _topk_body,
        grid=(_B, jc),
        in_specs=[pl.BlockSpec((_TCB, 3), lambda b, j: (b * jc + j, 0)),
                  pl.BlockSpec((1, 3, _N), lambda b, j: (b, 0, 0))],
        out_specs=pl.BlockSpec((_TCB, _K), lambda b, j: (b * jc + j, 0)),
        out_shape=jax.ShapeDtypeStruct((_B * _C, _K), jnp.int32),
        interpret=_INTERPRET,
    )(cent_flat, coords_t)


def _sc_gather(p_flat, idx_flat):
    nrows = idx_flat.shape[0]
    rows_w = nrows // _NW
    chunk = min(_GCHUNK, rows_w)
    mesh = plsc.VectorSubcoreMesh(core_axis_name="c", subcore_axis_name="s")

    @functools.partial(
        pl.kernel,
        out_type=jax.ShapeDtypeStruct((nrows, _IN), jnp.float32),
        mesh=mesh,
        scratch_types=[
            pltpu.VMEM((chunk,), jnp.int32),
            pltpu.VMEM((chunk, _IN), jnp.float32),
            pltpu.SemaphoreType.DMA,
        ],
        compiler_params=pltpu.CompilerParams(use_tc_tiling_on_sc=False),
    )
    def gk(p_hbm, idx_hbm, out_hbm, idx_v, rows_v, sem):
        wid = lax.axis_index("s") * 2 + lax.axis_index("c")
        base = wid * rows_w

        def body(j, carry):
            off = base + j * chunk
            pltpu.sync_copy(idx_hbm.at[pl.ds(off, chunk)], idx_v)
            pltpu.async_copy(p_hbm.at[idx_v], rows_v, sem).wait()
            pltpu.sync_copy(rows_v, out_hbm.at[pl.ds(off, chunk)])
            return carry

        lax.fori_loop(0, rows_w // chunk, body, 0)

    return gk(p_flat, idx_flat)


def _layer1(x_ref, cent_ref, w0_ref, b0_ref):
    y1 = jnp.dot(x_ref[...], w0_ref[...], preferred_element_type=jnp.float32) + b0_ref[...]
    cw = jnp.dot(cent_ref[...], w0_ref[0:3, :], preferred_element_type=jnp.float32)
    y1 = (y1.reshape(_TC, _K, y1.shape[-1]) - cw[:, None, :]).reshape(_TM, y1.shape[-1])
    return y1


def _acc_stats(i, s_ref, q_ref, y):
    s = jnp.sum(y, axis=0, keepdims=True)
    q = jnp.sum(y * y, axis=0, keepdims=True)

    @pl.when(i == 0)
    def _():
        s_ref[...] = jnp.zeros_like(s_ref)
        q_ref[...] = jnp.zeros_like(q_ref)

    s_ref[...] += s
    q_ref[...] += q


def _p1_kernel(x_ref, cent_ref, w0_ref, b0_ref, s_ref, q_ref):
    y1 = _layer1(x_ref, cent_ref, w0_ref, b0_ref)
    _acc_stats(pl.program_id(0), s_ref, q_ref, y1)


def _p2_kernel(x_ref, cent_ref, w0_ref, b0_ref, a1_ref, c1_ref, w1_ref, b1_ref,
               s_ref, q_ref):
    y1 = _layer1(x_ref, cent_ref, w0_ref, b0_ref)
    h1 = jnp.maximum(y1 * a1_ref[...] + c1_ref[...], 0.0)
    y2 = jnp.dot(h1, w1_ref[...], preferred_element_type=jnp.float32) + b1_ref[...]
    _acc_stats(pl.program_id(0), s_ref, q_ref, y2)


def _p3_kernel(x_ref, cent_ref, w0_ref, b0_ref, a1_ref, c1_ref, w1_ref, b1_ref,
               a2_ref, c2_ref, w2_ref, b2_ref, mx_ref, mn_ref, s_ref, q_ref):
    y1 = _layer1(x_ref, cent_ref, w0_ref, b0_ref)
    h1 = jnp.maximum(y1 * a1_ref[...] + c1_ref[...], 0.0)
    y2 = jnp.dot(h1, w1_ref[...], preferred_element_type=jnp.float32) + b1_ref[...]
    h2 = jnp.maximum(y2 * a2_ref[...] + c2_ref[...], 0.0)
    y3 = jnp.dot(h2, w2_ref[...], preferred_element_type=jnp.float32) + b2_ref[...]
    y3r = y3.reshape(_TC, _K, y3.shape[-1])
    mx_ref[...] = jnp.max(y3r, axis=1)
    mn_ref[...] = jnp.min(y3r, axis=1)
    _acc_stats(pl.program_id(0), s_ref, q_ref, y3)


def _p4_kernel(mx_ref, mn_ref, a_ref, c_ref, o_ref):
    a = a_ref[...]
    y = jnp.where(a >= 0.0, mx_ref[...], mn_ref[...]) * a + c_ref[...]
    o_ref[...] = jnp.maximum(y, 0.0)


def _row_spec(ch):
    return pl.BlockSpec((1, ch), lambda i: (0, 0))


def _full_spec(r, c):
    return pl.BlockSpec((r, c), lambda i: (0, 0))


def _stats_out(ch):
    return (jax.ShapeDtypeStruct((1, ch), jnp.float32),
            jax.ShapeDtypeStruct((1, ch), jnp.float32))


def _bn_affine(s, q, g, be):
    mu = s / _M
    var = q / _M - mu * mu
    a = g[None, :] / jnp.sqrt(var + _EPS)
    c = be[None, :] - mu * a
    return a, c


def kernel(coords, features, W0, b0, g0, be0, W1, b1, g1, be1, W2, b2, g2, be2):
    p_flat = jnp.concatenate([coords, features], axis=-1).reshape(_B * _N, _IN)
    gidx_cent = _fps_idx(jax.lax.stop_gradient(coords))
    cent_flat = _sc_gather(p_flat, gidx_cent)[:, :3]
    cent = cent_flat.reshape(_B, _C, 3)

    idx_flat = _ball_topk(cent_flat, jnp.transpose(coords, (0, 2, 1))).reshape(_M)
    x = _sc_gather(p_flat, idx_flat)
    w0t = W0.T
    w1t = W1.T
    w2t = W2.T
    b0r = b0[None, :]
    b1r = b1[None, :]
    b2r = b2[None, :]

    m1, m2, m3 = 64, 64, 128
    x_spec = pl.BlockSpec((_TM, _IN), lambda i: (i, 0))
    cent_spec = pl.BlockSpec((_TC, 3), lambda i: (i, 0))

    s1, q1 = pl.pallas_call(
        _p1_kernel,
        grid=(_GM,),
        in_specs=[x_spec, cent_spec, _full_spec(_IN, m1), _row_spec(m1)],
        out_specs=[_row_spec(m1), _row_spec(m1)],
        out_shape=_stats_out(m1),
        interpret=_INTERPRET,
    )(x, cent_flat, w0t, b0r)
    a1, c1 = _bn_affine(s1, q1, g0, be0)

    s2, q2 = pl.pallas_call(
        _p2_kernel,
        grid=(_GM,),
        in_specs=[x_spec, cent_spec, _full_spec(_IN, m1), _row_spec(m1),
                  _row_spec(m1), _row_spec(m1), _full_spec(m1, m2), _row_spec(m2)],
        out_specs=[_row_spec(m2), _row_spec(m2)],
        out_shape=_stats_out(m2),
        interpret=_INTERPRET,
    )(x, cent_flat, w0t, b0r, a1, c1, w1t, b1r)
    a2, c2 = _bn_affine(s2, q2, g1, be1)

    rows_out = _M // _K
    mx, mn, s3, q3 = pl.pallas_call(
        _p3_kernel,
        grid=(_GM,),
        in_specs=[x_spec, cent_spec, _full_spec(_IN, m1), _row_spec(m1),
                  _row_spec(m1), _row_spec(m1), _full_spec(m1, m2), _row_spec(m2),
                  _row_spec(m2), _row_spec(m2), _full_spec(m2, m3), _row_spec(m3)],
        out_specs=[pl.BlockSpec((_TC, m3), lambda i: (i, 0)),
                   pl.BlockSpec((_TC, m3), lambda i: (i, 0)),
                   _row_spec(m3), _row_spec(m3)],
        out_shape=(jax.ShapeDtypeStruct((rows_out, m3), jnp.float32),
                   jax.ShapeDtypeStruct((rows_out, m3), jnp.float32)) + _stats_out(m3),
        interpret=_INTERPRET,
    )(x, cent_flat, w0t, b0r, a1, c1, w1t, b1r, a2, c2, w2t, b2r)
    a3, c3 = _bn_affine(s3, q3, g2, be2)

    to = 2048
    out = pl.pallas_call(
        _p4_kernel,
        grid=(rows_out // to,),
        in_specs=[pl.BlockSpec((to, m3), lambda i: (i, 0)),
                  pl.BlockSpec((to, m3), lambda i: (i, 0)),
                  _row_spec(m3), _row_spec(m3)],
        out_specs=pl.BlockSpec((to, m3), lambda i: (i, 0)),
        out_shape=jax.ShapeDtypeStruct((rows_out, m3), jnp.float32),
        interpret=_INTERPRET,
    )(mx, mn, a3, c3)

    return (cent, out.reshape(_B, _C, m3))


# speedup vs baseline: 1.0269x; 1.0269x over previous
import functools

import jax
import jax.numpy as jnp
import numpy as np
from jax import lax
from jax.experimental import pallas as pl
from jax.experimental.pallas import tpu as pltpu
from jax.experimental.pallas import tpu_sc as plsc

_B = 16
_N = 4096
_D = 64
_C = 1024
_K = 32
_R2 = 0.2 * 0.2
_IN = 3 + _D
_INP = 72
_M = _B * _C * _K

_TM = 8192
_GM = _M // _TM
_TC = _TM // _K
_EPS = 1e-5

_NW = 32
_ROWS_W = _M // _NW
_GCHUNK = 512

_INTERPRET = False


def _fps_body(ct_ref, far0_ref, out_ref):
    ct = ct_ref[...]
    lin = jax.lax.broadcasted_iota(jnp.int32, (_B, _N), 1)
    big = jnp.int32(2 ** 30)

    offs = jax.lax.broadcasted_iota(jnp.int32, (1, _B), 1) * _N

    def body(i, carry):
        dist, far = carry
        out_ref[pl.ds(i, 1), :] = far.reshape(1, _B) + offs
        onehot = (lin == far).astype(jnp.float32)
        cd = [jnp.sum(onehot * ct[:, d, :], axis=1, keepdims=True)[:, None, :]
              for d in range(3)]
        dd = ct - jnp.concatenate(cd, axis=1)
        d2 = jnp.sum(dd * dd, axis=1)
        dist = jnp.minimum(dist, d2)
        m = jnp.max(dist, axis=1, keepdims=True)
        far = jnp.min(jnp.where(dist == m, lin, big), axis=1, keepdims=True)
        return dist, far

    jax.lax.fori_loop(
        0, _C, body,
        (jnp.full((_B, _N), jnp.float32(jnp.inf)), far0_ref[...]))


def _fps_idx(coords):
    farthest0 = jax.random.randint(jax.random.key(42), (_B,), 0, _N, dtype=jnp.int32)
    out = pl.pallas_call(
        _fps_body,
        grid=(1,),
        in_specs=[pl.BlockSpec((_B, 3, _N), lambda i: (0, 0, 0)),
                  pl.BlockSpec((_B, 1), lambda i: (0, 0))],
        out_specs=pl.BlockSpec((_C, _B), lambda i: (0, 0)),
        out_shape=jax.ShapeDtypeStruct((_C, _B), jnp.int32),
        interpret=_INTERPRET,
    )(jnp.transpose(coords, (0, 2, 1)), farthest0[:, None])
    return out.T.reshape(_B * _C)


_TCB = 256


def _topk_body(cent_ref, ct_ref, out_ref):
    ct = ct_ref[0]
    cent = cent_ref[...]
    x2 = jnp.sum(ct * ct, axis=0, keepdims=True)
    c2 = jnp.sum(cent * cent, axis=1, keepdims=True)
    dots = jnp.dot(cent, ct, preferred_element_type=jnp.float32)
    d2 = jnp.maximum(c2 + x2 - 2.0 * dots, 0.0)
    vals0 = jnp.where(d2 <= _R2, d2, jnp.float32(1e30))
    lin = jax.lax.broadcasted_iota(jnp.int32, (_TCB, _N), 1)
    kl = jax.lax.broadcasted_iota(jnp.int32, (_TCB, _K), 1)
    big = jnp.int32(2 ** 30)

    def body(k, carry):
        vals, acc, prev = carry
        vals = jnp.where(lin == prev, jnp.float32(1e31), vals)
        m = jnp.min(vals, axis=1, keepdims=True)
        idx = jnp.min(jnp.where(vals == m, lin, big), axis=1, keepdims=True)
        acc = jnp.where(kl == k, idx, acc)
        return vals, acc, idx

    _, acc, _ = jax.lax.fori_loop(
        0, _K, body,
        (vals0, jnp.zeros((_TCB, _K), jnp.int32),
         jnp.full((_TCB, 1), jnp.int32(-1))))
    out_ref[...] = acc + pl.program_id(0) * _N


def _ball_topk(cent_flat, coords_t):
    jc = _C // _TCB
    return pl.pallas_call(
        _topk_body,
        grid=(_B, jc),
        in_specs=[pl.BlockSpec((_TCB, 3), lambda b, j: (b * jc + j, 0)),
                  pl.BlockSpec((1, 3, _N), lambda b, j: (b, 0, 0))],
        out_specs=pl.BlockSpec((_TCB, _K), lambda b, j: (b * jc + j, 0)),
        out_shape=jax.ShapeDtypeStruct((_B * _C, _K), jnp.int32),
        interpret=_INTERPRET,
    )(cent_flat, coords_t)


def _sc_gather(p_flat, idx_flat):
    nrows = idx_flat.shape[0]
    rows_w = nrows // _NW
    chunk = min(_GCHUNK, rows_w)
    mesh = plsc.VectorSubcoreMesh(core_axis_name="c", subcore_axis_name="s")

    @functools.partial(
        pl.kernel,
        out_type=jax.ShapeDtypeStruct((nrows, _INP), jnp.float32),
        mesh=mesh,
        scratch_types=[
            pltpu.VMEM((chunk,), jnp.int32),
            pltpu.VMEM((chunk, _INP), jnp.float32),
            pltpu.SemaphoreType.DMA,
        ],
        compiler_params=pltpu.CompilerParams(use_tc_tiling_on_sc=False),
    )
    def gk(p_hbm, idx_hbm, out_hbm, idx_v, rows_v, sem):
        wid = lax.axis_index("s") * 2 + lax.axis_index("c")
        base = wid * rows_w

        def body(j, carry):
            off = base + j * chunk
            pltpu.sync_copy(idx_hbm.at[pl.ds(off, chunk)], idx_v)
            pltpu.async_copy(p_hbm.at[idx_v], rows_v, sem).wait()
            pltpu.sync_copy(rows_v, out_hbm.at[pl.ds(off, chunk)])
            return carry

        lax.fori_loop(0, rows_w // chunk, body, 0)

    return gk(p_flat, idx_flat)


def _layer1(x_ref, cent_ref, w0_ref, b0_ref):
    y1 = jnp.dot(x_ref[...], w0_ref[...], preferred_element_type=jnp.float32) + b0_ref[...]
    cw = jnp.dot(cent_ref[...], w0_ref[0:3, :], preferred_element_type=jnp.float32)
    y1 = (y1.reshape(_TC, _K, y1.shape[-1]) - cw[:, None, :]).reshape(_TM, y1.shape[-1])
    return y1


def _acc_stats(i, s_ref, q_ref, y):
    s = jnp.sum(y, axis=0, keepdims=True)
    q = jnp.sum(y * y, axis=0, keepdims=True)

    @pl.when(i == 0)
    def _():
        s_ref[...] = jnp.zeros_like(s_ref)
        q_ref[...] = jnp.zeros_like(q_ref)

    s_ref[...] += s
    q_ref[...] += q


def _p1_kernel(x_ref, cent_ref, w0_ref, b0_ref, s_ref, q_ref):
    y1 = _layer1(x_ref, cent_ref, w0_ref, b0_ref)
    _acc_stats(pl.program_id(0), s_ref, q_ref, y1)


def _p2_kernel(x_ref, cent_ref, w0_ref, b0_ref, a1_ref, c1_ref, w1_ref, b1_ref,
               s_ref, q_ref):
    y1 = _layer1(x_ref, cent_ref, w0_ref, b0_ref)
    h1 = jnp.maximum(y1 * a1_ref[...] + c1_ref[...], 0.0)
    y2 = jnp.dot(h1, w1_ref[...], preferred_element_type=jnp.float32) + b1_ref[...]
    _acc_stats(pl.program_id(0), s_ref, q_ref, y2)


def _p3_kernel(x_ref, cent_ref, w0_ref, b0_ref, a1_ref, c1_ref, w1_ref, b1_ref,
               a2_ref, c2_ref, w2_ref, b2_ref, mx_ref, mn_ref, s_ref, q_ref):
    y1 = _layer1(x_ref, cent_ref, w0_ref, b0_ref)
    h1 = jnp.maximum(y1 * a1_ref[...] + c1_ref[...], 0.0)
    y2 = jnp.dot(h1, w1_ref[...], preferred_element_type=jnp.float32) + b1_ref[...]
    h2 = jnp.maximum(y2 * a2_ref[...] + c2_ref[...], 0.0)
    y3 = jnp.dot(h2, w2_ref[...], preferred_element_type=jnp.float32) + b2_ref[...]
    y3r = y3.reshape(_TC, _K, y3.shape[-1])
    mx_ref[...] = jnp.max(y3r, axis=1)
    mn_ref[...] = jnp.min(y3r, axis=1)
    _acc_stats(pl.program_id(0), s_ref, q_ref, y3)


def _p4_kernel(mx_ref, mn_ref, a_ref, c_ref, o_ref):
    a = a_ref[...]
    y = jnp.where(a >= 0.0, mx_ref[...], mn_ref[...]) * a + c_ref[...]
    o_ref[...] = jnp.maximum(y, 0.0)


def _row_spec(ch):
    return pl.BlockSpec((1, ch), lambda i: (0, 0))


def _full_spec(r, c):
    return pl.BlockSpec((r, c), lambda i: (0, 0))


def _stats_out(ch):
    return (jax.ShapeDtypeStruct((1, ch), jnp.float32),
            jax.ShapeDtypeStruct((1, ch), jnp.float32))


def _bn_affine(s, q, g, be):
    mu = s / _M
    var = q / _M - mu * mu
    a = g[None, :] / jnp.sqrt(var + _EPS)
    c = be[None, :] - mu * a
    return a, c


def kernel(coords, features, W0, b0, g0, be0, W1, b1, g1, be1, W2, b2, g2, be2):
    p_flat = jnp.concatenate(
        [coords, features,
         jnp.zeros((_B, _N, _INP - _IN), jnp.float32)], axis=-1).reshape(_B * _N, _INP)
    gidx_cent = _fps_idx(jax.lax.stop_gradient(coords))
    cent_flat = _sc_gather(p_flat, gidx_cent)[:, :3]
    cent = cent_flat.reshape(_B, _C, 3)

    idx_flat = _ball_topk(cent_flat, jnp.transpose(coords, (0, 2, 1))).reshape(_M)
    x = _sc_gather(p_flat, idx_flat)
    w0t = jnp.concatenate([W0.T, jnp.zeros((_INP - _IN, W0.shape[0]), jnp.float32)])
    w1t = W1.T
    w2t = W2.T
    b0r = b0[None, :]
    b1r = b1[None, :]
    b2r = b2[None, :]

    m1, m2, m3 = 64, 64, 128
    x_spec = pl.BlockSpec((_TM, _INP), lambda i: (i, 0))
    cent_spec = pl.BlockSpec((_TC, 3), lambda i: (i, 0))

    s1, q1 = pl.pallas_call(
        _p1_kernel,
        grid=(_GM,),
        in_specs=[x_spec, cent_spec, _full_spec(_INP, m1), _row_spec(m1)],
        out_specs=[_row_spec(m1), _row_spec(m1)],
        out_shape=_stats_out(m1),
        interpret=_INTERPRET,
    )(x, cent_flat, w0t, b0r)
    a1, c1 = _bn_affine(s1, q1, g0, be0)

    s2, q2 = pl.pallas_call(
        _p2_kernel,
        grid=(_GM,),
        in_specs=[x_spec, cent_spec, _full_spec(_INP, m1), _row_spec(m1),
                  _row_spec(m1), _row_spec(m1), _full_spec(m1, m2), _row_spec(m2)],
        out_specs=[_row_spec(m2), _row_spec(m2)],
        out_shape=_stats_out(m2),
        interpret=_INTERPRET,
    )(x, cent_flat, w0t, b0r, a1, c1, w1t, b1r)
    a2, c2 = _bn_affine(s2, q2, g1, be1)

    rows_out = _M // _K
    mx, mn, s3, q3 = pl.pallas_call(
        _p3_kernel,
        grid=(_GM,),
        in_specs=[x_spec, cent_spec, _full_spec(_INP, m1), _row_spec(m1),
                  _row_spec(m1), _row_spec(m1), _full_spec(m1, m2), _row_spec(m2),
                  _row_spec(m2), _row_spec(m2), _full_spec(m2, m3), _row_spec(m3)],
        out_specs=[pl.BlockSpec((_TC, m3), lambda i: (i, 0)),
                   pl.BlockSpec((_TC, m3), lambda i: (i, 0)),
                   _row_spec(m3), _row_spec(m3)],
        out_shape=(jax.ShapeDtypeStruct((rows_out, m3), jnp.float32),
                   jax.ShapeDtypeStruct((rows_out, m3), jnp.float32)) + _stats_out(m3),
        interpret=_INTERPRET,
    )(x, cent_flat, w0t, b0r, a1, c1, w1t, b1r, a2, c2, w2t, b2r)
    a3, c3 = _bn_affine(s3, q3, g2, be2)

    to = 2048
    out = pl.pallas_call(
        _p4_kernel,
        grid=(rows_out // to,),
        in_specs=[pl.BlockSpec((to, m3), lambda i: (i, 0)),
                  pl.BlockSpec((to, m3), lambda i: (i, 0)),
                  _row_spec(m3), _row_spec(m3)],
        out_specs=pl.BlockSpec((to, m3), lambda i: (i, 0)),
        out_shape=jax.ShapeDtypeStruct((rows_out, m3), jnp.float32),
        interpret=_INTERPRET,
    )(mx, mn, a3, c3)

    return (cent, out.reshape(_B, _C, m3))
